# baseline (speedup 1.0000x reference)
import functools

import jax
import jax.numpy as jnp
from jax import lax
from jax.experimental import pallas as pl
from jax.experimental.pallas import tpu as pltpu
from jax.experimental.pallas import tpu_sc as plsc

LATENT_DIM = 32
BATCH = 16384

_info = plsc.get_sparse_core_info()
_NC, _NS = _info.num_cores, _info.num_subcores
_NW = _NC * _NS
_B_PER_W = BATCH // _NW

_mesh = plsc.VectorSubcoreMesh(core_axis_name="c", subcore_axis_name="s")


@functools.partial(
    pl.kernel,
    mesh=_mesh,
    out_type=(
        jax.ShapeDtypeStruct((BATCH, LATENT_DIM), jnp.float32),
        jax.ShapeDtypeStruct((BATCH, LATENT_DIM), jnp.float32),
    ),
    scratch_types=[
        pltpu.VMEM((_B_PER_W,), jnp.int32),
        pltpu.VMEM((_B_PER_W, LATENT_DIM), jnp.float32),
        pltpu.VMEM((_B_PER_W,), jnp.int32),
        pltpu.VMEM((_B_PER_W, LATENT_DIM), jnp.float32),
        pltpu.SemaphoreType.DMA,
        pltpu.SemaphoreType.DMA,
    ],
    compiler_params=pltpu.CompilerParams(use_tc_tiling_on_sc=False),
)
def _dual_gather(user_ids_hbm, item_ids_hbm, user_table_hbm, item_table_hbm,
                 user_out_hbm, item_out_hbm,
                 uidx_v, urows_v, iidx_v, irows_v, usem, isem):
    wid = lax.axis_index("s") * _NC + lax.axis_index("c")
    base = wid * _B_PER_W
    pltpu.sync_copy(user_ids_hbm.at[pl.ds(base, _B_PER_W)], uidx_v)
    pltpu.sync_copy(item_ids_hbm.at[pl.ds(base, _B_PER_W)], iidx_v)
    ucopy = pltpu.async_copy(user_table_hbm.at[uidx_v], urows_v, usem)
    icopy = pltpu.async_copy(item_table_hbm.at[iidx_v], irows_v, isem)
    ucopy.wait()
    pltpu.sync_copy(urows_v, user_out_hbm.at[pl.ds(base, _B_PER_W)])
    icopy.wait()
    pltpu.sync_copy(irows_v, item_out_hbm.at[pl.ds(base, _B_PER_W)])


def kernel(user_ids, item_ids, user_table, item_table):
    return _dual_gather(user_ids, item_ids, user_table, item_table)
